# baseline (speedup 1.0000x reference)
import functools

import jax
import jax.numpy as jnp
from jax import lax
from jax.experimental import pallas as pl
from jax.experimental.pallas import tpu as pltpu
from jax.experimental.pallas import tpu_sc as plsc

LLM_D = 2048
VIT_D = 1024
POOL = 64
TOPK = 8
ALPHA = 16
IN_F = 2048
OUT_F = 2048
TOK = 8192

TILE = 512
SCALING = ALPHA / TOPK
K_RATIO = VIT_D / LLM_D

LANES = 16
KEYS_PER_TILE = 2


def _sc_scores(kl, kv, ql, qv):
    mesh = plsc.VectorSubcoreMesh(core_axis_name="c", subcore_axis_name="s")

    @functools.partial(
        pl.kernel,
        mesh=mesh,
        out_type=jax.ShapeDtypeStruct((POOL, LANES), jnp.float32),
        scratch_types=[
            pltpu.VMEM((KEYS_PER_TILE, LLM_D), jnp.float32),
            pltpu.VMEM((KEYS_PER_TILE, VIT_D), jnp.float32),
            pltpu.VMEM((LLM_D,), jnp.float32),
            pltpu.VMEM((VIT_D,), jnp.float32),
            pltpu.VMEM((KEYS_PER_TILE, LANES), jnp.float32),
        ],
    )
    def sc_kernel(kl_hbm, kv_hbm, ql_hbm, qv_hbm, o_hbm,
                  kl_v, kv_v, ql_v, qv_v, res_v):
        wid = lax.axis_index("s") * 2 + lax.axis_index("c")
        base = wid * KEYS_PER_TILE
        pltpu.sync_copy(kl_hbm.at[pl.ds(base, KEYS_PER_TILE), :], kl_v)
        pltpu.sync_copy(kv_hbm.at[pl.ds(base, KEYS_PER_TILE), :], kv_v)
        pltpu.sync_copy(ql_hbm, ql_v)
        pltpu.sync_copy(qv_hbm, qv_v)

        for kk in range(KEYS_PER_TILE):
            kl_row = kl_v.at[kk]
            kv_row = kv_v.at[kk]

            def llm_body(i, acc):
                j = i * LANES
                return acc + kl_row[pl.ds(j, LANES)] * ql_v[pl.ds(j, LANES)]

            def vit_body(i, acc):
                j = i * LANES
                return acc + kv_row[pl.ds(j, LANES)] * qv_v[pl.ds(j, LANES)]

            acc = lax.fori_loop(0, LLM_D // LANES, llm_body,
                                jnp.zeros((LANES,), jnp.float32))
            accv = lax.fori_loop(0, VIT_D // LANES, vit_body,
                                 jnp.zeros((LANES,), jnp.float32))
            res_row = res_v.at[kk]
            res_row[...] = acc + K_RATIO * accv

        pltpu.sync_copy(res_v, o_hbm.at[pl.ds(base, KEYS_PER_TILE), :])

    return sc_kernel(kl, kv, ql, qv)


def _fused_kernel(x_ref, sacc_ref, a_ref, b_pool_ref, w_ref, bias_ref,
                  o_ref, weff_ref):
    @pl.when(pl.program_id(0) == 0)
    def _fold():
        s_vec = jnp.sum(sacc_ref[...], axis=1)
        s_row = jnp.reshape(s_vec, (1, POOL))
        s_col = jnp.reshape(s_vec, (POOL, 1))
        j_idx = jax.lax.broadcasted_iota(jnp.int32, (POOL, POOL), 1)
        k_idx = jax.lax.broadcasted_iota(jnp.int32, (POOL, POOL), 0)
        beats = (s_row > s_col) | ((s_row == s_col) & (j_idx < k_idx))
        rank = jnp.sum(beats.astype(jnp.int32), axis=1, keepdims=True)
        mask = (rank < TOPK).astype(jnp.float32)
        a_m = a_ref[...] * (jnp.reshape(mask, (1, POOL)) * SCALING)
        delta = jax.lax.dot_general(b_pool_ref[...], a_m,
                                    (((0,), (1,)), ((), ())))
        weff_ref[...] = (w_ref[...] + delta).astype(jnp.bfloat16)

    out = jax.lax.dot_general(x_ref[...], weff_ref[...],
                              (((1,), (1,)), ((), ())),
                              preferred_element_type=jnp.float32)
    o_ref[...] = out + bias_ref[...]


@jax.jit
def kernel(x, llm_query, vit_query, static_keys_llm, static_keys_vit,
           A_pool, B_pool, W, b):
    bias = jnp.reshape(b, (1, OUT_F))

    sacc = _sc_scores(static_keys_llm, static_keys_vit, llm_query, vit_query)

    full = lambda shape: pl.BlockSpec(shape, lambda i: (0, 0))
    return pl.pallas_call(
        _fused_kernel,
        grid=(TOK // TILE,),
        in_specs=[
            pl.BlockSpec((TILE, IN_F), lambda i: (i, 0)),
            full((POOL, LANES)),
            full((IN_F, POOL)),
            full((POOL, OUT_F)),
            full((OUT_F, IN_F)),
            full((1, OUT_F)),
        ],
        out_specs=pl.BlockSpec((TILE, OUT_F), lambda i: (i, 0)),
        out_shape=jax.ShapeDtypeStruct((TOK, OUT_F), jnp.float32),
        scratch_shapes=[pltpu.VMEM((OUT_F, IN_F), jnp.bfloat16)],
        compiler_params=pltpu.CompilerParams(
            dimension_semantics=("arbitrary",),
        ),
    )(x, sacc, A_pool, B_pool, W, bias)


# speedup vs baseline: 1.2428x; 1.2428x over previous
import jax
import jax.numpy as jnp
from jax.experimental import pallas as pl
from jax.experimental.pallas import tpu as pltpu

LLM_D = 2048
VIT_D = 1024
POOL = 64
TOPK = 8
ALPHA = 16
IN_F = 2048
OUT_F = 2048
TOK = 8192

TILE = 512
SCALING = ALPHA / TOPK
K_RATIO = VIT_D / LLM_D


def _fused_kernel(x_ref, ql_ref, qv_ref, kl_ref, kv_ref, a_ref, b_pool_ref,
                  w_ref, bias_ref, o_ref, weff_ref):
    @pl.when(pl.program_id(0) == 0)
    def _fold():
        hi = jax.lax.Precision.HIGHEST
        s_llm = jax.lax.dot_general(ql_ref[...], kl_ref[...],
                                    (((1,), (1,)), ((), ())), precision=hi)
        s_vit = jax.lax.dot_general(qv_ref[...], kv_ref[...],
                                    (((1,), (1,)), ((), ())), precision=hi)
        s_row = s_llm + K_RATIO * s_vit
        s_col = jnp.reshape(s_row, (POOL, 1))
        j_idx = jax.lax.broadcasted_iota(jnp.int32, (POOL, POOL), 1)
        k_idx = jax.lax.broadcasted_iota(jnp.int32, (POOL, POOL), 0)
        beats = (s_row > s_col) | ((s_row == s_col) & (j_idx < k_idx))
        rank = jnp.sum(beats.astype(jnp.int32), axis=1, keepdims=True)
        mask = (rank < TOPK).astype(jnp.float32)
        a_m = (a_ref[...] * (jnp.reshape(mask, (1, POOL)) * SCALING)
               ).astype(jnp.bfloat16)
        delta = jax.lax.dot_general(b_pool_ref[...].astype(jnp.bfloat16), a_m,
                                    (((0,), (1,)), ((), ())),
                                    preferred_element_type=jnp.float32)
        weff_ref[...] = (w_ref[...] + delta).astype(jnp.bfloat16)

    out = jax.lax.dot_general(x_ref[...], weff_ref[...],
                              (((1,), (1,)), ((), ())),
                              preferred_element_type=jnp.float32)
    o_ref[...] = out + bias_ref[...]


@jax.jit
def kernel(x, llm_query, vit_query, static_keys_llm, static_keys_vit,
           A_pool, B_pool, W, b):
    ql = jnp.reshape(llm_query, (1, LLM_D))
    qv = jnp.reshape(vit_query, (1, VIT_D))
    bias = jnp.reshape(b, (1, OUT_F))

    full = lambda shape: pl.BlockSpec(shape, lambda i: (0, 0))
    return pl.pallas_call(
        _fused_kernel,
        grid=(TOK // TILE,),
        in_specs=[
            pl.BlockSpec((TILE, IN_F), lambda i: (i, 0)),
            full((1, LLM_D)),
            full((1, VIT_D)),
            full((POOL, LLM_D)),
            full((POOL, VIT_D)),
            full((IN_F, POOL)),
            full((POOL, OUT_F)),
            full((OUT_F, IN_F)),
            full((1, OUT_F)),
        ],
        out_specs=pl.BlockSpec((TILE, OUT_F), lambda i: (i, 0)),
        out_shape=jax.ShapeDtypeStruct((TOK, OUT_F), jnp.float32),
        scratch_shapes=[pltpu.VMEM((OUT_F, IN_F), jnp.bfloat16)],
        compiler_params=pltpu.CompilerParams(
            dimension_semantics=("arbitrary",),
        ),
    )(x, ql, qv, static_keys_llm, static_keys_vit, A_pool, B_pool, W, bias)
